# baseline (speedup 1.0000x reference)
import functools
import math

import jax
import jax.numpy as jnp
from jax import lax
from jax.experimental import pallas as pl
from jax.experimental.pallas import tpu as pltpu
from jax.experimental.pallas import tpu_sc as plsc

_N_FEATS = 26
_VOCAB = 100000
_D_EMBED = 32
_D_MODEL = 1024
_B, _L = 1024, 50
_NTOK = _B * _L
_P = _N_FEATS * _D_EMBED
_HALF = _P // 2

_NW = 32
_TCH = 6400
_NTCH = _NTOK // _TCH
assert _NTOK % _TCH == 0 and _TCH % 256 == 0


def _sc_gather_body(t_hbm, samp_hbm, out_hbm, vec_v, idx_v, out_v,
                    isems, osems, *, row0, nrows):
    rows_per_tile = nrows // _NW
    wid = lax.axis_index("s") * 2 + lax.axis_index("c")
    p0 = row0 + wid * rows_per_tile

    def prow(i, _):
        p = p0 + i
        f = p // _D_EMBED
        pltpu.async_copy(samp_hbm.at[f, pl.ds(0, _TCH)], idx_v.at[0], isems[0])
        pltpu.sync_copy(t_hbm.at[p], vec_v)

        @pl.when(i > 0)
        def _():
            for b in range(2):
                pltpu.make_async_copy(
                    out_v.at[b], out_hbm.at[0, pl.ds(0, _TCH)],
                    osems[b]).wait()

        for c in range(_NTCH):
            b = c % 2
            if c + 1 < _NTCH:
                pltpu.async_copy(samp_hbm.at[f, pl.ds((c + 1) * _TCH, _TCH)],
                                 idx_v.at[1 - b], isems[1 - b])
            pltpu.make_async_copy(
                samp_hbm.at[f, pl.ds(c * _TCH, _TCH)], idx_v.at[b],
                isems[b]).wait()

            @plsc.parallel_loop(0, _TCH, 16, unroll=16)
            def _gather(s, b=b):
                iv = idx_v[b, pl.ds(s, 16)]
                out_v[b, pl.ds(s, 16)] = plsc.load_gather(vec_v, [iv])
            pltpu.async_copy(
                out_v.at[b], out_hbm.at[p - row0, pl.ds(c * _TCH, _TCH)],
                osems[b])
        return ()

    lax.fori_loop(0, rows_per_tile, prow, ())
    for b in range(2):
        pltpu.make_async_copy(
            out_v.at[b], out_hbm.at[0, pl.ds(0, _TCH)], osems[b]).wait()


@functools.cache
def _sc_gather(row0, nrows):
    return pl.kernel(
        functools.partial(_sc_gather_body, row0=row0, nrows=nrows),
        out_type=jax.ShapeDtypeStruct((nrows, _NTOK), jnp.float32),
        mesh=plsc.VectorSubcoreMesh(core_axis_name="c", subcore_axis_name="s"),
        scratch_types=[
            pltpu.VMEM((_VOCAB,), jnp.float32),
            pltpu.VMEM((2, _TCH), jnp.int32),
            pltpu.VMEM((2, _TCH), jnp.float32),
            [pltpu.SemaphoreType.DMA, pltpu.SemaphoreType.DMA],
            [pltpu.SemaphoreType.DMA, pltpu.SemaphoreType.DMA],
        ],
        compiler_params=pltpu.CompilerParams(
            use_tc_tiling_on_sc=True, needs_layout_passes=False),
    )


_TM = 1024


def _proj_first_body(embt_ref, samp_ref, wt_ref, et_ref, o_ref):
    mask_t = (samp_ref[...] != 0).astype(jnp.bfloat16)
    m_t = lax.dot_general(
        et_ref[...], mask_t, (((1,), (0,)), ((), ())),
        preferred_element_type=jnp.float32)
    xm = (embt_ref[...] * m_t).astype(jnp.bfloat16)
    o_ref[...] = lax.dot_general(
        xm, wt_ref[...], (((0,), (0,)), ((), ())),
        preferred_element_type=jnp.float32)


def _proj_second_body(part_ref, embt_ref, samp_ref, wt_ref, et_ref, b_ref,
                      o_ref):
    mask_t = (samp_ref[...] != 0).astype(jnp.bfloat16)
    m_t = lax.dot_general(
        et_ref[...], mask_t, (((1,), (0,)), ((), ())),
        preferred_element_type=jnp.float32)
    xm = (embt_ref[...] * m_t).astype(jnp.bfloat16)
    acc = lax.dot_general(
        xm, wt_ref[...], (((0,), (0,)), ((), ())),
        preferred_element_type=jnp.float32)
    o_ref[...] = part_ref[...] + acc + b_ref[...]


def _projection(embt_a, embt_b, samp_f, wt_a, wt_b, et_a, et_b, b_row):
    grid = (_NTOK // _TM,)
    common = dict(
        out_specs=pl.BlockSpec((_TM, _D_MODEL), lambda i: (i, 0)),
        out_shape=jax.ShapeDtypeStruct((_NTOK, _D_MODEL), jnp.float32),
        grid=grid,
    )
    embt_spec = pl.BlockSpec((_HALF, _TM), lambda i: (0, i))
    samp_spec = pl.BlockSpec((_N_FEATS, _TM), lambda i: (0, i))
    wt_spec = pl.BlockSpec((_HALF, _D_MODEL), lambda i: (0, 0))
    et_spec = pl.BlockSpec((_HALF, _N_FEATS), lambda i: (0, 0))
    part = pl.pallas_call(
        _proj_first_body,
        in_specs=[embt_spec, samp_spec, wt_spec, et_spec],
        **common,
    )(embt_a, samp_f, wt_a, et_a)
    return pl.pallas_call(
        _proj_second_body,
        in_specs=[
            pl.BlockSpec((_TM, _D_MODEL), lambda i: (i, 0)),
            embt_spec, samp_spec, wt_spec, et_spec,
            pl.BlockSpec((1, _D_MODEL), lambda i: (0, 0)),
        ],
        input_output_aliases={0: 0},
        **common,
    )(part, embt_b, samp_f, wt_b, et_b, b_row)


def kernel(sample, tables, W, b):
    t_flat = tables.transpose(0, 2, 1).reshape(_P, _VOCAB)
    samp_f = sample.transpose(2, 1, 0).reshape(_N_FEATS, _NTOK).astype(jnp.int32)

    embt_a = _sc_gather(0, _HALF)(t_flat, samp_f)
    embt_b = _sc_gather(_HALF, _HALF)(t_flat, samp_f)

    scale = math.sqrt(float(_D_MODEL))
    wt_bf = (W.T * scale).astype(jnp.bfloat16)
    et_bf = (jnp.arange(_P)[:, None] // _D_EMBED
             == jnp.arange(_N_FEATS)[None, :]).astype(jnp.bfloat16)
    b_row = (b * scale).reshape(1, _D_MODEL)

    out = _projection(embt_a, embt_b, samp_f,
                      wt_bf[:_HALF], wt_bf[_HALF:],
                      et_bf[:_HALF], et_bf[_HALF:], b_row)
    return out.reshape(_L, _B, _D_MODEL).transpose(1, 0, 2)


# speedup vs baseline: 1.0625x; 1.0625x over previous
import functools
import math

import jax
import jax.numpy as jnp
from jax import lax
from jax.experimental import pallas as pl
from jax.experimental.pallas import tpu as pltpu
from jax.experimental.pallas import tpu_sc as plsc

_N_FEATS = 26
_VOCAB = 100000
_D_EMBED = 32
_D_MODEL = 1024
_B, _L = 1024, 50
_NTOK = _B * _L
_P = _N_FEATS * _D_EMBED

_NW = 32
_ROWS_PER_TILE = _P // _NW
_TCH = 6400
_NTCH = _NTOK // _TCH
assert _NTOK % _TCH == 0 and _TCH % 256 == 0


def _sc_gather_body(t_hbm, samp_hbm, out_hbm, vec_v, idx_v, out_v,
                    isems, osems):
    wid = lax.axis_index("s") * 2 + lax.axis_index("c")
    p0 = wid * _ROWS_PER_TILE

    def prow(i, _):
        p = p0 + i
        f = p // _D_EMBED
        pltpu.async_copy(samp_hbm.at[f, pl.ds(0, _TCH)], idx_v.at[0], isems[0])
        pltpu.sync_copy(t_hbm.at[p], vec_v)

        @pl.when(i > 0)
        def _():
            for b in range(2):
                pltpu.make_async_copy(
                    out_v.at[b], out_hbm.at[p, pl.ds(0, _TCH)], osems[b]).wait()

        for c in range(_NTCH):
            b = c % 2
            if c + 1 < _NTCH:
                pltpu.async_copy(samp_hbm.at[f, pl.ds((c + 1) * _TCH, _TCH)],
                                 idx_v.at[1 - b], isems[1 - b])
            pltpu.make_async_copy(
                samp_hbm.at[f, pl.ds(c * _TCH, _TCH)], idx_v.at[b],
                isems[b]).wait()
            if c >= 2:
                pltpu.make_async_copy(
                    out_v.at[b], out_hbm.at[p, pl.ds((c - 2) * _TCH, _TCH)],
                    osems[b]).wait()

            @plsc.parallel_loop(0, _TCH, 16, unroll=32)
            def _gather(s, b=b):
                iv = idx_v[b, pl.ds(s, 16)]
                out_v[b, pl.ds(s, 16)] = plsc.load_gather(vec_v, [iv])
            pltpu.async_copy(out_v.at[b], out_hbm.at[p, pl.ds(c * _TCH, _TCH)],
                             osems[b])
        return ()

    lax.fori_loop(0, _ROWS_PER_TILE, prow, ())
    for b in range(2):
        pltpu.make_async_copy(
            out_v.at[b], out_hbm.at[0, pl.ds(0, _TCH)], osems[b]).wait()


@functools.cache
def _sc_gather():
    return pl.kernel(
        _sc_gather_body,
        out_type=jax.ShapeDtypeStruct((_P, _NTOK), jnp.float32),
        mesh=plsc.VectorSubcoreMesh(core_axis_name="c", subcore_axis_name="s"),
        scratch_types=[
            pltpu.VMEM((_VOCAB,), jnp.float32),
            pltpu.VMEM((2, _TCH), jnp.int32),
            pltpu.VMEM((2, _TCH), jnp.float32),
            [pltpu.SemaphoreType.DMA, pltpu.SemaphoreType.DMA],
            [pltpu.SemaphoreType.DMA, pltpu.SemaphoreType.DMA],
        ],
        compiler_params=pltpu.CompilerParams(
            use_tc_tiling_on_sc=True, needs_layout_passes=False),
    )


_TM = 1024


def _proj_body(embt_ref, samp_ref, wt_ref, et_ref, b_ref, o_ref):
    mask_t = (samp_ref[...] != 0).astype(jnp.bfloat16)
    m_t = lax.dot_general(
        et_ref[...], mask_t, (((1,), (0,)), ((), ())),
        preferred_element_type=jnp.float32)
    xm = (embt_ref[...] * m_t).astype(jnp.bfloat16)
    acc = lax.dot_general(
        xm, wt_ref[...], (((0,), (0,)), ((), ())),
        preferred_element_type=jnp.float32)
    o_ref[...] = acc + b_ref[...]


def _projection(embt, samp_u, wt_bf, et_bf, b_row):
    return pl.pallas_call(
        _proj_body,
        grid=(_NTOK // _TM,),
        in_specs=[
            pl.BlockSpec((_P, _TM), lambda i: (0, i)),
            pl.BlockSpec((_N_FEATS, _TM), lambda i: (0, i)),
            pl.BlockSpec((_P, _D_MODEL), lambda i: (0, 0)),
            pl.BlockSpec((_P, _N_FEATS), lambda i: (0, 0)),
            pl.BlockSpec((1, _D_MODEL), lambda i: (0, 0)),
        ],
        out_specs=pl.BlockSpec((_TM, _D_MODEL), lambda i: (i, 0)),
        out_shape=jax.ShapeDtypeStruct((_NTOK, _D_MODEL), jnp.float32),
    )(embt, samp_u, wt_bf, et_bf, b_row)


def kernel(sample, tables, W, b):
    t_flat = tables.transpose(0, 2, 1).reshape(_P, _VOCAB)
    samp_f = sample.transpose(2, 1, 0).reshape(_N_FEATS, _NTOK).astype(jnp.int32)

    embt = _sc_gather()(t_flat, samp_f)

    scale = math.sqrt(float(_D_MODEL))
    wt_bf = (W.T * scale).astype(jnp.bfloat16)
    et_bf = (jnp.arange(_P)[:, None] // _D_EMBED
             == jnp.arange(_N_FEATS)[None, :]).astype(jnp.bfloat16)
    b_row = (b * scale).reshape(1, _D_MODEL)

    out = _projection(embt, samp_f, wt_bf, et_bf, b_row)
    return out.reshape(_L, _B, _D_MODEL).transpose(1, 0, 2)


# speedup vs baseline: 1.1284x; 1.0620x over previous
import functools
import math

import jax
import jax.numpy as jnp
from jax import lax
from jax.experimental import pallas as pl
from jax.experimental.pallas import tpu as pltpu
from jax.experimental.pallas import tpu_sc as plsc

_N_FEATS = 26
_VOCAB = 100000
_D_EMBED = 32
_D_MODEL = 1024
_B, _L = 1024, 50
_NTOK = _B * _L
_P = _N_FEATS * _D_EMBED

_NW = 32
_ROWS_PER_TILE = _P // _NW
_TCH = 6400
_NTCH = _NTOK // _TCH
assert _NTOK % _TCH == 0 and _TCH % 256 == 0


def _sc_gather_body(t_hbm, samp_hbm, out_hbm, vec_v, idx_v, out_v,
                    isems, osems):
    wid = lax.axis_index("s") * 2 + lax.axis_index("c")
    p0 = wid * _ROWS_PER_TILE

    def prow(i, _):
        p = p0 + i
        f = p // _D_EMBED
        pltpu.async_copy(samp_hbm.at[f, pl.ds(0, _TCH)], idx_v.at[0], isems[0])
        pltpu.sync_copy(t_hbm.at[p], vec_v)

        @pl.when(i > 0)
        def _():
            for b in range(2):
                pltpu.make_async_copy(
                    out_v.at[b], out_hbm.at[p, pl.ds(0, _TCH)], osems[b]).wait()

        for c in range(_NTCH):
            b = c % 2
            if c + 1 < _NTCH:
                pltpu.async_copy(samp_hbm.at[f, pl.ds((c + 1) * _TCH, _TCH)],
                                 idx_v.at[1 - b], isems[1 - b])
            pltpu.make_async_copy(
                samp_hbm.at[f, pl.ds(c * _TCH, _TCH)], idx_v.at[b],
                isems[b]).wait()
            if c >= 2:
                pltpu.make_async_copy(
                    out_v.at[b], out_hbm.at[p, pl.ds((c - 2) * _TCH, _TCH)],
                    osems[b]).wait()

            @plsc.parallel_loop(0, _TCH, 16, unroll=16)
            def _gather(s, b=b):
                iv = idx_v[b, pl.ds(s, 16)]
                out_v[b, pl.ds(s, 16)] = plsc.load_gather(vec_v, [iv])
            pltpu.async_copy(out_v.at[b], out_hbm.at[p, pl.ds(c * _TCH, _TCH)],
                             osems[b])
        return ()

    lax.fori_loop(0, _ROWS_PER_TILE, prow, ())
    for b in range(2):
        pltpu.make_async_copy(
            out_v.at[b], out_hbm.at[0, pl.ds(0, _TCH)], osems[b]).wait()


@functools.cache
def _sc_gather():
    return pl.kernel(
        _sc_gather_body,
        out_type=jax.ShapeDtypeStruct((_P, _NTOK), jnp.float32),
        mesh=plsc.VectorSubcoreMesh(core_axis_name="c", subcore_axis_name="s"),
        scratch_types=[
            pltpu.VMEM((_VOCAB,), jnp.float32),
            pltpu.VMEM((2, _TCH), jnp.int32),
            pltpu.VMEM((2, _TCH), jnp.float32),
            [pltpu.SemaphoreType.DMA, pltpu.SemaphoreType.DMA],
            [pltpu.SemaphoreType.DMA, pltpu.SemaphoreType.DMA],
        ],
        compiler_params=pltpu.CompilerParams(
            use_tc_tiling_on_sc=True, needs_layout_passes=False),
    )


_TM = 1024


def _proj_body(embt_ref, samp_ref, wt_ref, et_ref, b_ref, o_ref):
    mask_t = (samp_ref[...] != 0).astype(jnp.bfloat16)
    m_t = lax.dot_general(
        et_ref[...], mask_t, (((1,), (0,)), ((), ())),
        preferred_element_type=jnp.float32)
    xm = (embt_ref[...] * m_t).astype(jnp.bfloat16)
    acc = lax.dot_general(
        xm, wt_ref[...], (((0,), (0,)), ((), ())),
        preferred_element_type=jnp.float32)
    o_ref[...] = acc + b_ref[...]


def _projection(embt, samp_u, wt_bf, et_bf, b_row):
    return pl.pallas_call(
        _proj_body,
        grid=(_NTOK // _TM,),
        in_specs=[
            pl.BlockSpec((_P, _TM), lambda i: (0, i)),
            pl.BlockSpec((_N_FEATS, _TM), lambda i: (0, i)),
            pl.BlockSpec((_P, _D_MODEL), lambda i: (0, 0)),
            pl.BlockSpec((_P, _N_FEATS), lambda i: (0, 0)),
            pl.BlockSpec((1, _D_MODEL), lambda i: (0, 0)),
        ],
        out_specs=pl.BlockSpec((_TM, _D_MODEL), lambda i: (i, 0)),
        out_shape=jax.ShapeDtypeStruct((_NTOK, _D_MODEL), jnp.float32),
    )(embt, samp_u, wt_bf, et_bf, b_row)


def kernel(sample, tables, W, b):
    t_flat = tables.transpose(0, 2, 1).reshape(_P, _VOCAB)
    samp_f = sample.transpose(2, 1, 0).reshape(_N_FEATS, _NTOK).astype(jnp.int32)

    embt = _sc_gather()(t_flat, samp_f)

    scale = math.sqrt(float(_D_MODEL))
    wt_bf = (W.T * scale).astype(jnp.bfloat16)
    et_bf = (jnp.arange(_P)[:, None] // _D_EMBED
             == jnp.arange(_N_FEATS)[None, :]).astype(jnp.bfloat16)
    b_row = (b * scale).reshape(1, _D_MODEL)

    out = _projection(embt, samp_f, wt_bf, et_bf, b_row)
    return out.reshape(_L, _B, _D_MODEL).transpose(1, 0, 2)
